# baseline (speedup 1.0000x reference)
import jax
import jax.numpy as jnp
from jax.experimental import pallas as pl
from jax.experimental.pallas import tpu as pltpu

T = 256
H = 1024
E = 16
K = 2
I = 1024


def _router_kernel(x_ref, wq_ref, cmb_ref):
    x = x_ref[...]
    mix = jax.lax.dot_general(
        x, wq_ref[...], (((1,), (1,)), ((), ())),
        preferred_element_type=jnp.float32)
    q = mix[:, 0:E]
    k = mix[:, E:2 * E]
    v = mix[:, 2 * E:3 * E]
    aw = q[:, :, None] * k[:, None, :]
    m = jnp.max(aw, axis=-1, keepdims=True)
    ex = jnp.exp(aw - m)
    s = jnp.sum(ex, axis=-1)
    num = jnp.sum(ex * v[:, None, :], axis=-1)
    logits = num / s
    iota = jax.lax.broadcasted_iota(jnp.int32, (T, E), 1)
    m1 = jnp.max(logits, axis=-1, keepdims=True)
    a1 = jnp.min(jnp.where(logits == m1, iota, E), axis=-1, keepdims=True)
    masked = jnp.where(iota == a1, -jnp.inf, logits)
    m2 = jnp.max(masked, axis=-1, keepdims=True)
    a2 = jnp.min(jnp.where(masked == m2, iota, E), axis=-1, keepdims=True)
    w1 = jax.nn.sigmoid(m1 - m2)
    w2 = 1.0 - w1
    oh1 = (iota == a1).astype(jnp.float32)
    oh2 = (iota == a2).astype(jnp.float32)
    cmb_ref[...] = oh1 * w1 + oh2 * w2


def _expert_kernel(x_ref, cmb_ref, w1_ref, w2_ref, o_ref):
    e = pl.program_id(0)
    x = x_ref[...].astype(jnp.bfloat16)
    a = jax.lax.dot_general(
        x, w1_ref[0], (((1,), (1,)), ((), ())),
        preferred_element_type=jnp.float32)
    gate = a[:, :I]
    up = a[:, I:]
    h = (gate * jax.nn.sigmoid(gate) * up).astype(jnp.bfloat16)
    y = jax.lax.dot_general(
        h, w2_ref[0], (((1,), (1,)), ((), ())),
        preferred_element_type=jnp.float32)
    oh = (jax.lax.broadcasted_iota(jnp.int32, (E, 1), 0) == e
          ).astype(jnp.float32)
    col = jax.lax.dot_general(
        cmb_ref[...], oh, (((1,), (0,)), ((), ())),
        preferred_element_type=jnp.float32)
    contrib = col * y

    @pl.when(e == 0)
    def _():
        o_ref[...] = contrib

    @pl.when(e > 0)
    def _():
        o_ref[...] += contrib


def kernel(hidden_states, w_qkv, w1, w2):
    combine = pl.pallas_call(
        _router_kernel,
        out_shape=jax.ShapeDtypeStruct((T, E), jnp.float32),
    )(hidden_states, w_qkv)

    w1 = w1.astype(jnp.bfloat16)
    w2 = w2.astype(jnp.bfloat16)
    out = pl.pallas_call(
        _expert_kernel,
        grid=(E,),
        in_specs=[
            pl.BlockSpec((T, H), lambda e: (0, 0)),
            pl.BlockSpec((T, E), lambda e: (0, 0)),
            pl.BlockSpec((1, 2 * I, H), lambda e: (e, 0, 0)),
            pl.BlockSpec((1, H, I), lambda e: (e, 0, 0)),
        ],
        out_specs=pl.BlockSpec((T, H), lambda e: (0, 0)),
        out_shape=jax.ShapeDtypeStruct((T, H), jnp.float32),
        compiler_params=pltpu.CompilerParams(
            dimension_semantics=("arbitrary",)),
    )(hidden_states, combine, w1, w2)
    return out


# speedup vs baseline: 2.0345x; 2.0345x over previous
import jax
import jax.numpy as jnp
from jax.experimental import pallas as pl
from jax.experimental.pallas import tpu as pltpu

T = 256
H = 1024
E = 16
K = 2
I = 1024


def _router(x, wq):
    mix = jax.lax.dot_general(
        x, wq, (((1,), (1,)), ((), ())),
        preferred_element_type=jnp.float32)
    q = mix[:, 0:E]
    k = mix[:, E:2 * E]
    v = mix[:, 2 * E:3 * E]
    aw = q[:, :, None] * k[:, None, :]
    m = jnp.max(aw, axis=-1, keepdims=True)
    ex = jnp.exp(aw - m)
    s = jnp.sum(ex, axis=-1)
    num = jnp.sum(ex * v[:, None, :], axis=-1)
    logits = num / s
    iota = jax.lax.broadcasted_iota(jnp.int32, (T, E), 1)
    m1 = jnp.max(logits, axis=-1, keepdims=True)
    a1 = jnp.min(jnp.where(logits == m1, iota, E), axis=-1, keepdims=True)
    masked = jnp.where(iota == a1, -jnp.inf, logits)
    m2 = jnp.max(masked, axis=-1, keepdims=True)
    a2 = jnp.min(jnp.where(masked == m2, iota, E), axis=-1, keepdims=True)
    w1 = jax.nn.sigmoid(m1 - m2)
    w2 = 1.0 - w1
    oh1 = (iota == a1).astype(jnp.float32)
    oh2 = (iota == a2).astype(jnp.float32)
    return oh1 * w1 + oh2 * w2


def _moe_kernel(x_ref, wq_ref, w1_ref, w2_ref, o_ref, cmb_ref):
    e = pl.program_id(0)

    @pl.when(e == 0)
    def _():
        cmb_ref[...] = _router(x_ref[...], wq_ref[...])

    x = x_ref[...].astype(jnp.bfloat16)
    a = jax.lax.dot_general(
        x, w1_ref[0].astype(jnp.bfloat16), (((1,), (1,)), ((), ())),
        preferred_element_type=jnp.float32)
    gate = a[:, :I]
    up = a[:, I:]
    h = (gate * jax.nn.sigmoid(gate) * up).astype(jnp.bfloat16)
    y = jax.lax.dot_general(
        h, w2_ref[0].astype(jnp.bfloat16), (((1,), (1,)), ((), ())),
        preferred_element_type=jnp.float32)
    oh = (jax.lax.broadcasted_iota(jnp.int32, (E, 1), 0) == e
          ).astype(jnp.float32)
    col = jax.lax.dot_general(
        cmb_ref[...], oh, (((1,), (0,)), ((), ())),
        preferred_element_type=jnp.float32)
    contrib = col * y

    @pl.when(e == 0)
    def _():
        o_ref[...] = contrib

    @pl.when(e > 0)
    def _():
        o_ref[...] += contrib


def kernel(hidden_states, w_qkv, w1, w2):
    out = pl.pallas_call(
        _moe_kernel,
        grid=(E,),
        in_specs=[
            pl.BlockSpec((T, H), lambda e: (0, 0)),
            pl.BlockSpec((3 * E, H), lambda e: (0, 0)),
            pl.BlockSpec((1, 2 * I, H), lambda e: (e, 0, 0)),
            pl.BlockSpec((1, H, I), lambda e: (e, 0, 0)),
        ],
        out_specs=pl.BlockSpec((T, H), lambda e: (0, 0)),
        out_shape=jax.ShapeDtypeStruct((T, H), jnp.float32),
        scratch_shapes=[pltpu.VMEM((T, E), jnp.float32)],
        compiler_params=pltpu.CompilerParams(
            dimension_semantics=("arbitrary",)),
    )(hidden_states, w_qkv, w1, w2)
    return out
